# baseline (speedup 1.0000x reference)
import jax
import jax.numpy as jnp
from jax import lax
from jax.experimental import pallas as pl
from jax.experimental.pallas import tpu as pltpu

_D = 768
_DFF = 1536
_E = 8
_NTOK = 2048
_BT = 256


def _moe_dense_body(x_ref, gw_ref, gb_ref, W1_ref, b1_ref, W2_ref, b2_ref, out_ref):
    e = pl.program_id(1)
    x = x_ref[...]

    logits = jnp.dot(x, gw_ref[...], preferred_element_type=jnp.float32) + gb_ref[...]
    iota = lax.broadcasted_iota(jnp.int32, (_BT, _E), 1)
    m1 = jnp.max(logits, axis=1, keepdims=True)
    i1 = jnp.min(jnp.where(logits >= m1, iota, _E), axis=1, keepdims=True)
    l2 = jnp.where(iota == i1, -jnp.inf, logits)
    m2 = jnp.max(l2, axis=1, keepdims=True)
    i2 = jnp.min(jnp.where(l2 >= m2, iota, _E), axis=1, keepdims=True)
    z = jnp.sum(jnp.exp(logits - m1), axis=1, keepdims=True)
    p1 = 1.0 / z
    p2 = jnp.exp(m2 - m1) / z
    t = jnp.exp(p2 - p1)
    w1 = 1.0 / (1.0 + t)
    w2 = t / (1.0 + t)
    ce = jnp.where(i1 == e, w1, 0.0) + jnp.where(i2 == e, w2, 0.0)

    h = jnp.maximum(jnp.dot(x, W1_ref[0], preferred_element_type=jnp.float32) + b1_ref[0], 0.0)
    y = jnp.dot(h, W2_ref[0], preferred_element_type=jnp.float32) + b2_ref[0]

    @pl.when(e == 0)
    def _():
        out_ref[...] = jnp.zeros_like(out_ref)

    out_ref[...] += ce * y


def kernel(x, gate_w, gate_b, W1, b1, W2, b2):
    return pl.pallas_call(
        _moe_dense_body,
        grid=(_NTOK // _BT, _E),
        in_specs=[
            pl.BlockSpec((_BT, _D), lambda n, e: (n, 0)),
            pl.BlockSpec((_D, _E), lambda n, e: (0, 0)),
            pl.BlockSpec((1, _E), lambda n, e: (0, 0)),
            pl.BlockSpec((1, _D, _DFF), lambda n, e: (e, 0, 0)),
            pl.BlockSpec((1, 1, _DFF), lambda n, e: (e, 0, 0)),
            pl.BlockSpec((1, _DFF, _D), lambda n, e: (e, 0, 0)),
            pl.BlockSpec((1, 1, _D), lambda n, e: (e, 0, 0)),
        ],
        out_specs=pl.BlockSpec((_BT, _D), lambda n, e: (n, 0)),
        out_shape=jax.ShapeDtypeStruct((_NTOK, _D), jnp.float32),
        compiler_params=pltpu.CompilerParams(
            dimension_semantics=("parallel", "arbitrary"),
        ),
    )(x, gate_w, gate_b.reshape(1, _E), W1, b1.reshape(_E, 1, _DFF), W2, b2.reshape(_E, 1, _D))
